# Initial kernel scaffold; baseline (speedup 1.0000x reference)
import jax
import jax.numpy as jnp
from jax.experimental import pallas as pl


def kernel(x, edge_index, edge_weight, edge_attr, W1, b1, gamma, beta, W2, b2):
    raise NotImplementedError("write your pallas kernel here")


# speedup vs baseline: 2.2005x; 2.2005x over previous
import jax
import jax.numpy as jnp
from jax import lax
from jax.experimental import pallas as pl
from jax.experimental.pallas import tpu as pltpu
from jax.experimental.pallas import tpu_sc as plsc

N = 10000
E = 320000
D = 128
ED = 16
OUT = 128
BN_EPS = 1e-5

NC = 2
NS = 16
NW = NC * NS
EPT = E // NW
K = 80
NCHUNK = EPT // K
RPT = 624
RREM = N - NS * RPT


def _sc_body(row_h, col_h, w_h, attr_h, x_h, zx_h, po_h, pe_h,
             acc, colv, rowv, wv, attrv, xg, pay, gsem):
  c = lax.axis_index("c")
  s = lax.axis_index("s")
  r0 = s * RPT
  base = (c * NS + s) * EPT
  lane = lax.iota(jnp.int32, 16)

  def zero_acc():
    pltpu.sync_copy(zx_h.at[pl.ds(r0, RPT), :], acc.at[pl.ds(r0, RPT), :])

    @pl.when(s == 0)
    def _zrem():
      pltpu.sync_copy(zx_h.at[pl.ds(NS * RPT, RREM), :],
                      acc.at[pl.ds(NS * RPT, RREM), :])

  def drain_acc(out_h):
    pltpu.sync_copy(acc.at[pl.ds(r0, RPT), :], out_h.at[c, pl.ds(r0, RPT), :])

    @pl.when(s == 0)
    def _drem():
      pltpu.sync_copy(acc.at[pl.ds(NS * RPT, RREM), :],
                      out_h.at[c, pl.ds(NS * RPT, RREM), :])

  zero_acc()
  plsc.subcore_barrier()

  def chunk_a(i, carry):
    off = base + i * K
    pltpu.sync_copy(col_h.at[pl.ds(off, K)], colv)
    pltpu.sync_copy(row_h.at[pl.ds(off, K)], rowv)
    pltpu.sync_copy(w_h.at[pl.ds(off, K)], wv)
    pltpu.async_copy(x_h.at[colv], xg, gsem).wait()

    def edge(e, cc):
      wspl = plsc.load_gather(wv, [jnp.full((16,), e, jnp.int32)])
      for j in range(8):
        xg[e, pl.ds(j * 16, 16)] = xg[e, pl.ds(j * 16, 16)] * wspl
      return cc

    lax.fori_loop(0, K, edge, 0)
    pltpu.sync_copy(xg, acc.at[rowv], add=True)
    return carry

  lax.fori_loop(0, NCHUNK, chunk_a, 0)
  plsc.subcore_barrier()
  drain_acc(po_h)
  plsc.subcore_barrier()

  zero_acc()
  z16 = jnp.zeros((16,), jnp.float32)

  def zpay(e, cc):
    for j in range(2, 8):
      pay[e, pl.ds(j * 16, 16)] = z16
    return cc

  lax.fori_loop(0, K, zpay, 0)
  plsc.subcore_barrier()

  def chunk_b(i, carry):
    off = base + i * K
    pltpu.sync_copy(row_h.at[pl.ds(off, K)], rowv)
    pltpu.sync_copy(w_h.at[pl.ds(off, K)], wv)
    pltpu.sync_copy(attr_h.at[pl.ds(off, K), :], attrv)

    def edge(e, cc):
      wspl = plsc.load_gather(wv, [jnp.full((16,), e, jnp.int32)])
      pay[e, pl.ds(0, 16)] = attrv[e, :] * wspl
      pay[e, pl.ds(16, 16)] = jnp.where(lane == 0, wspl, 0.0)
      return cc

    lax.fori_loop(0, K, edge, 0)
    pltpu.sync_copy(pay, acc.at[rowv], add=True)
    return carry

  lax.fori_loop(0, NCHUNK, chunk_b, 0)
  plsc.subcore_barrier()
  drain_acc(pe_h)


_sc_agg = pl.kernel(
    _sc_body,
    out_type=(
        jax.ShapeDtypeStruct((NC, N, D), jnp.float32),
        jax.ShapeDtypeStruct((NC, N, D), jnp.float32),
    ),
    mesh=plsc.VectorSubcoreMesh(core_axis_name="c", subcore_axis_name="s"),
    compiler_params=pltpu.CompilerParams(needs_layout_passes=False),
    scratch_types=[
        pltpu.VMEM_SHARED((N, D), jnp.float32),
        pltpu.VMEM((K,), jnp.int32),
        pltpu.VMEM((K,), jnp.int32),
        pltpu.VMEM((K,), jnp.float32),
        pltpu.VMEM((K, ED), jnp.float32),
        pltpu.VMEM((K, D), jnp.float32),
        pltpu.VMEM((K, D), jnp.float32),
        pltpu.SemaphoreType.DMA,
    ],
)


def _mlp_body(x_ref, po_ref, pe_ref, w1a_ref, w1b_ref, w1c_ref,
              b1_ref, g_ref, b_ref, w2_ref, b2_ref, out_ref):
  xo = po_ref[0] + po_ref[1]
  ed = pe_ref[0] + pe_ref[1]
  ae = ed[:, :ED]
  deg = ed[:, ED:ED + 1]
  xs = deg * x_ref[...]
  h = (jnp.dot(xs, w1a_ref[...], preferred_element_type=jnp.float32)
       + jnp.dot(xo, w1b_ref[...], preferred_element_type=jnp.float32)
       + jnp.dot(ae, w1c_ref[...], preferred_element_type=jnp.float32)
       + b1_ref[...])
  h = jnp.maximum(h, 0.0)
  mean = jnp.mean(h, axis=0, keepdims=True)
  hc = h - mean
  var = jnp.mean(hc * hc, axis=0, keepdims=True)
  hn = hc * lax.rsqrt(var + BN_EPS) * g_ref[...] + b_ref[...]
  out_ref[...] = (jnp.dot(hn, w2_ref[...], preferred_element_type=jnp.float32)
                  + b2_ref[...])


_mlp = pl.pallas_call(
    _mlp_body,
    out_shape=jax.ShapeDtypeStruct((N, OUT), jnp.float32),
)


def kernel(x, edge_index, edge_weight, edge_attr, W1, b1, gamma, beta, W2, b2):
  row = edge_index[0]
  col = edge_index[1]
  zx = jnp.zeros((N, D), jnp.float32)
  po, pe = _sc_agg(row, col, edge_weight, edge_attr, x, zx)
  w1a, w1b, w1c = W1[:D], W1[D:2 * D], W1[2 * D:]
  return _mlp(x, po, pe, w1a, w1b, w1c,
              b1.reshape(1, OUT), gamma.reshape(1, OUT), beta.reshape(1, OUT),
              W2, b2.reshape(1, OUT))


# speedup vs baseline: 2.2480x; 1.0216x over previous
import jax
import jax.numpy as jnp
from jax import lax
from jax.experimental import pallas as pl
from jax.experimental.pallas import tpu as pltpu
from jax.experimental.pallas import tpu_sc as plsc

N = 10000
E = 320000
D = 128
ED = 16
OUT = 128
BN_EPS = 1e-5

NC = 2
NS = 16
NW = NC * NS
EPT = E // NW
K = 80
U = 8
NCHUNK = EPT // K
RPT = 624
RREM = N - NS * RPT


def _sc_body(row_h, col_h, w_h, attr_h, x_h, zx_h, po_h, pe_h,
             acc, colv, rowv, wv, attrv, xg, pay, gsem):
  c = lax.axis_index("c")
  s = lax.axis_index("s")
  r0 = s * RPT
  base = (c * NS + s) * EPT
  lane = lax.iota(jnp.int32, 16)

  def zero_acc():
    pltpu.sync_copy(zx_h.at[pl.ds(r0, RPT), :], acc.at[pl.ds(r0, RPT), :])

    @pl.when(s == 0)
    def _zrem():
      pltpu.sync_copy(zx_h.at[pl.ds(NS * RPT, RREM), :],
                      acc.at[pl.ds(NS * RPT, RREM), :])

  def drain_acc(out_h):
    pltpu.sync_copy(acc.at[pl.ds(r0, RPT), :], out_h.at[c, pl.ds(r0, RPT), :])

    @pl.when(s == 0)
    def _drem():
      pltpu.sync_copy(acc.at[pl.ds(NS * RPT, RREM), :],
                      out_h.at[c, pl.ds(NS * RPT, RREM), :])

  zero_acc()
  plsc.subcore_barrier()

  def chunk_a(i, carry):
    off = base + i * K
    pltpu.sync_copy(col_h.at[pl.ds(off, K)], colv)
    pltpu.sync_copy(row_h.at[pl.ds(off, K)], rowv)
    pltpu.sync_copy(w_h.at[pl.ds(off, K)], wv)
    pltpu.async_copy(x_h.at[colv], xg, gsem).wait()

    def group(g, cc):
      for u in range(U):
        e = g * U + u
        wspl = plsc.load_gather(wv, [jnp.full((16,), e, jnp.int32)])
        for j in range(8):
          xg[e, pl.ds(j * 16, 16)] = xg[e, pl.ds(j * 16, 16)] * wspl
      return cc

    lax.fori_loop(0, K // U, group, 0)
    pltpu.sync_copy(xg, acc.at[rowv], add=True)
    return carry

  lax.fori_loop(0, NCHUNK, chunk_a, 0)
  plsc.subcore_barrier()
  drain_acc(po_h)
  plsc.subcore_barrier()

  zero_acc()
  z16 = jnp.zeros((16,), jnp.float32)

  def zpay(e, cc):
    for j in range(2, 8):
      pay[e, pl.ds(j * 16, 16)] = z16
    return cc

  lax.fori_loop(0, K, zpay, 0)
  plsc.subcore_barrier()

  def chunk_b(i, carry):
    off = base + i * K
    pltpu.sync_copy(row_h.at[pl.ds(off, K)], rowv)
    pltpu.sync_copy(w_h.at[pl.ds(off, K)], wv)
    pltpu.sync_copy(attr_h.at[pl.ds(off, K), :], attrv)

    def group(g, cc):
      for u in range(U):
        e = g * U + u
        wspl = plsc.load_gather(wv, [jnp.full((16,), e, jnp.int32)])
        pay[e, pl.ds(0, 16)] = attrv[e, :] * wspl
        pay[e, pl.ds(16, 16)] = jnp.where(lane == 0, wspl, 0.0)
      return cc

    lax.fori_loop(0, K // U, group, 0)
    pltpu.sync_copy(pay, acc.at[rowv], add=True)
    return carry

  lax.fori_loop(0, NCHUNK, chunk_b, 0)
  plsc.subcore_barrier()
  drain_acc(pe_h)


_sc_agg = pl.kernel(
    _sc_body,
    out_type=(
        jax.ShapeDtypeStruct((NC, N, D), jnp.float32),
        jax.ShapeDtypeStruct((NC, N, D), jnp.float32),
    ),
    mesh=plsc.VectorSubcoreMesh(core_axis_name="c", subcore_axis_name="s"),
    compiler_params=pltpu.CompilerParams(needs_layout_passes=False),
    scratch_types=[
        pltpu.VMEM_SHARED((N, D), jnp.float32),
        pltpu.VMEM((K,), jnp.int32),
        pltpu.VMEM((K,), jnp.int32),
        pltpu.VMEM((K,), jnp.float32),
        pltpu.VMEM((K, ED), jnp.float32),
        pltpu.VMEM((K, D), jnp.float32),
        pltpu.VMEM((K, D), jnp.float32),
        pltpu.SemaphoreType.DMA,
    ],
)


def _mlp_body(x_ref, po_ref, pe_ref, w1a_ref, w1b_ref, w1c_ref,
              b1_ref, g_ref, b_ref, w2_ref, b2_ref, out_ref):
  xo = po_ref[0] + po_ref[1]
  ed = pe_ref[0] + pe_ref[1]
  ae = ed[:, :ED]
  deg = ed[:, ED:ED + 1]
  xs = deg * x_ref[...]
  h = (jnp.dot(xs, w1a_ref[...], preferred_element_type=jnp.float32)
       + jnp.dot(xo, w1b_ref[...], preferred_element_type=jnp.float32)
       + jnp.dot(ae, w1c_ref[...], preferred_element_type=jnp.float32)
       + b1_ref[...])
  h = jnp.maximum(h, 0.0)
  mean = jnp.mean(h, axis=0, keepdims=True)
  hc = h - mean
  var = jnp.mean(hc * hc, axis=0, keepdims=True)
  hn = hc * lax.rsqrt(var + BN_EPS) * g_ref[...] + b_ref[...]
  out_ref[...] = (jnp.dot(hn, w2_ref[...], preferred_element_type=jnp.float32)
                  + b2_ref[...])


_mlp = pl.pallas_call(
    _mlp_body,
    out_shape=jax.ShapeDtypeStruct((N, OUT), jnp.float32),
)


def kernel(x, edge_index, edge_weight, edge_attr, W1, b1, gamma, beta, W2, b2):
  row = edge_index[0]
  col = edge_index[1]
  zx = jnp.zeros((N, D), jnp.float32)
  po, pe = _sc_agg(row, col, edge_weight, edge_attr, x, zx)
  w1a, w1b, w1c = W1[:D], W1[D:2 * D], W1[2 * D:]
  return _mlp(x, po, pe, w1a, w1b, w1c,
              b1.reshape(1, OUT), gamma.reshape(1, OUT), beta.reshape(1, OUT),
              W2, b2.reshape(1, OUT))


# speedup vs baseline: 2.5975x; 1.1555x over previous
import jax
import jax.numpy as jnp
from jax import lax
from jax.experimental import pallas as pl
from jax.experimental.pallas import tpu as pltpu
from jax.experimental.pallas import tpu_sc as plsc

N = 10000
E = 320000
D = 128
ED = 16
OUT = 128
BN_EPS = 1e-5

NC = 2
NS = 16
NW = NC * NS
EPT = E // NW
K = 80
U = 8
NCHUNK = EPT // K
RPT = 624
RREM = N - NS * RPT


def _sc_body(row_h, col_h, w_h, attr_h, x_h, zx_h, po_h, pe_h,
             acc, colv, colv2, rowv, wv, attrv, xg, xg2, pay, gsem, gsem2):
  c = lax.axis_index("c")
  s = lax.axis_index("s")
  r0 = s * RPT
  base = (c * NS + s) * EPT
  lane = lax.iota(jnp.int32, 16)

  def zero_acc():
    pltpu.sync_copy(zx_h.at[pl.ds(r0, RPT), :], acc.at[pl.ds(r0, RPT), :])

    @pl.when(s == 0)
    def _zrem():
      pltpu.sync_copy(zx_h.at[pl.ds(NS * RPT, RREM), :],
                      acc.at[pl.ds(NS * RPT, RREM), :])

  def drain_acc(out_h):
    pltpu.sync_copy(acc.at[pl.ds(r0, RPT), :], out_h.at[c, pl.ds(r0, RPT), :])

    @pl.when(s == 0)
    def _drem():
      pltpu.sync_copy(acc.at[pl.ds(NS * RPT, RREM), :],
                      out_h.at[c, pl.ds(NS * RPT, RREM), :])

  zero_acc()
  plsc.subcore_barrier()

  def prefetch(i, xgb, colb, sem):
    off = base + i * K
    pltpu.sync_copy(col_h.at[pl.ds(off, K)], colb)
    return pltpu.async_copy(x_h.at[colb], xgb, sem)

  def work_a(i, xgb):
    off = base + i * K
    pltpu.sync_copy(row_h.at[pl.ds(off, K)], rowv)
    pltpu.sync_copy(w_h.at[pl.ds(off, K)], wv)

    def group(g, cc):
      for u in range(U):
        e = g * U + u
        wspl = plsc.load_gather(wv, [jnp.full((16,), e, jnp.int32)])
        for j in range(8):
          xgb[e, pl.ds(j * 16, 16)] = xgb[e, pl.ds(j * 16, 16)] * wspl
      return cc

    lax.fori_loop(0, K // U, group, 0)
    pltpu.sync_copy(xgb, acc.at[rowv], add=True)

  def wait_gather(xgb, colb, sem):
    pltpu.make_async_copy(x_h.at[colb], xgb, sem).wait()

  prefetch(0, xg, colv, gsem)

  def chunk_pair(p, carry):
    i = 2 * p
    prefetch(i + 1, xg2, colv2, gsem2)
    wait_gather(xg, colv, gsem)
    work_a(i, xg)
    prefetch(i + 2, xg, colv, gsem)
    wait_gather(xg2, colv2, gsem2)
    work_a(i + 1, xg2)
    return carry

  lax.fori_loop(0, NCHUNK // 2, chunk_pair, 0)
  wait_gather(xg, colv, gsem)
  work_a(NCHUNK - 1, xg)
  plsc.subcore_barrier()
  drain_acc(po_h)
  plsc.subcore_barrier()

  zero_acc()
  z16 = jnp.zeros((16,), jnp.float32)

  def zpay(e, cc):
    for j in range(2, 8):
      pay[e, pl.ds(j * 16, 16)] = z16
    return cc

  lax.fori_loop(0, K, zpay, 0)
  plsc.subcore_barrier()

  def chunk_b(i, carry):
    off = base + i * K
    pltpu.sync_copy(row_h.at[pl.ds(off, K)], rowv)
    pltpu.sync_copy(w_h.at[pl.ds(off, K)], wv)
    pltpu.sync_copy(attr_h.at[pl.ds(off, K), :], attrv)

    def group(g, cc):
      for u in range(U):
        e = g * U + u
        wspl = plsc.load_gather(wv, [jnp.full((16,), e, jnp.int32)])
        pay[e, pl.ds(0, 16)] = attrv[e, :] * wspl
        pay[e, pl.ds(16, 16)] = jnp.where(lane == 0, wspl, 0.0)
      return cc

    lax.fori_loop(0, K // U, group, 0)
    pltpu.sync_copy(pay, acc.at[rowv], add=True)
    return carry

  lax.fori_loop(0, NCHUNK, chunk_b, 0)
  plsc.subcore_barrier()
  drain_acc(pe_h)


_sc_agg = pl.kernel(
    _sc_body,
    out_type=(
        jax.ShapeDtypeStruct((NC, N, D), jnp.float32),
        jax.ShapeDtypeStruct((NC, N, D), jnp.float32),
    ),
    mesh=plsc.VectorSubcoreMesh(core_axis_name="c", subcore_axis_name="s"),
    compiler_params=pltpu.CompilerParams(needs_layout_passes=False),
    scratch_types=[
        pltpu.VMEM_SHARED((N, D), jnp.float32),
        pltpu.VMEM((K,), jnp.int32),
        pltpu.VMEM((K,), jnp.int32),
        pltpu.VMEM((K,), jnp.int32),
        pltpu.VMEM((K,), jnp.float32),
        pltpu.VMEM((K, ED), jnp.float32),
        pltpu.VMEM((K, D), jnp.float32),
        pltpu.VMEM((K, D), jnp.float32),
        pltpu.VMEM((K, D), jnp.float32),
        pltpu.SemaphoreType.DMA,
        pltpu.SemaphoreType.DMA,
    ],
)


def _mlp_body(x_ref, po_ref, pe_ref, w1a_ref, w1b_ref, w1c_ref,
              b1_ref, g_ref, b_ref, w2_ref, b2_ref, out_ref):
  xo = po_ref[0] + po_ref[1]
  ed = pe_ref[0] + pe_ref[1]
  ae = ed[:, :ED]
  deg = ed[:, ED:ED + 1]
  xs = deg * x_ref[...]
  h = (jnp.dot(xs, w1a_ref[...], preferred_element_type=jnp.float32)
       + jnp.dot(xo, w1b_ref[...], preferred_element_type=jnp.float32)
       + jnp.dot(ae, w1c_ref[...], preferred_element_type=jnp.float32)
       + b1_ref[...])
  h = jnp.maximum(h, 0.0)
  mean = jnp.mean(h, axis=0, keepdims=True)
  hc = h - mean
  var = jnp.mean(hc * hc, axis=0, keepdims=True)
  hn = hc * lax.rsqrt(var + BN_EPS) * g_ref[...] + b_ref[...]
  out_ref[...] = (jnp.dot(hn, w2_ref[...], preferred_element_type=jnp.float32)
                  + b2_ref[...])


_mlp = pl.pallas_call(
    _mlp_body,
    out_shape=jax.ShapeDtypeStruct((N, OUT), jnp.float32),
)


def kernel(x, edge_index, edge_weight, edge_attr, W1, b1, gamma, beta, W2, b2):
  row = edge_index[0]
  col = edge_index[1]
  zx = jnp.zeros((N, D), jnp.float32)
  po, pe = _sc_agg(row, col, edge_weight, edge_attr, x, zx)
  w1a, w1b, w1c = W1[:D], W1[D:2 * D], W1[2 * D:]
  return _mlp(x, po, pe, w1a, w1b, w1c,
              b1.reshape(1, OUT), gamma.reshape(1, OUT), beta.reshape(1, OUT),
              W2, b2.reshape(1, OUT))
